# Initial kernel scaffold; baseline (speedup 1.0000x reference)
import jax
import jax.numpy as jnp
from jax.experimental import pallas as pl


def kernel(memory, last_update_t, src, dst, edge_feat, timestamps, time_w, time_b, w_ih, w_hh, b_ih, b_hh):
    raise NotImplementedError("write your pallas kernel here")


# speedup vs baseline: 1.5262x; 1.5262x over previous
import functools

import jax
import jax.numpy as jnp
from jax import lax
from jax.experimental import pallas as pl
from jax.experimental.pallas import tpu as pltpu
from jax.experimental.pallas import tpu_sc as plsc

N_NODES = 100000
D_MEM = 128
D_EDGE = 16
D_TIME = 16
B_EV = 16384

NC, NS = 2, 16
NW = NC * NS
BPW = B_EV // NW
CH = 128
NCH = BPW // CH

_MESH = plsc.VectorSubcoreMesh(core_axis_name="c", subcore_axis_name="s")


def _wid():
  return lax.axis_index("s") * NC + lax.axis_index("c")


@functools.partial(
    pl.kernel,
    mesh=_MESH,
    out_type=(
        jax.ShapeDtypeStruct((B_EV, D_MEM), jnp.float32),
        jax.ShapeDtypeStruct((B_EV, D_MEM), jnp.float32),
        jax.ShapeDtypeStruct((B_EV,), jnp.float32),
    ),
    scratch_types=[
        pltpu.VMEM((NCH, CH), jnp.int32),
        pltpu.VMEM((NCH, CH), jnp.int32),
        pltpu.VMEM((CH, D_MEM), jnp.float32),
        pltpu.VMEM((CH,), jnp.float32),
        pltpu.SemaphoreType.DMA,
    ],
)
def _sc_gather(mem_hbm, lut_hbm, srcr_hbm, dstr_hbm,
               srcm_out, dstm_out, lut_out,
               sidx, didx, rows, lutv, sem):
  wid = _wid()
  base = wid * BPW
  pltpu.sync_copy(srcr_hbm.at[wid], sidx)
  pltpu.sync_copy(dstr_hbm.at[wid], didx)
  for c in range(NCH):
    off = base + c * CH
    pltpu.async_copy(mem_hbm.at[sidx.at[c]], rows, sem).wait()
    pltpu.sync_copy(rows, srcm_out.at[pl.ds(off, CH)])
    pltpu.async_copy(mem_hbm.at[didx.at[c]], rows, sem).wait()
    pltpu.sync_copy(rows, dstm_out.at[pl.ds(off, CH)])
    pltpu.async_copy(lut_hbm.at[sidx.at[c]], lutv, sem).wait()
    pltpu.sync_copy(lutv, lut_out.at[pl.ds(off, CH)])


@functools.partial(
    pl.kernel,
    mesh=_MESH,
    out_type=(),
    scratch_types=[
        pltpu.VMEM((NCH, CH), jnp.int32),
        pltpu.VMEM((NCH, CH), jnp.int32),
        pltpu.VMEM((CH, D_MEM), jnp.float32),
        pltpu.VMEM((CH,), jnp.float32),
        pltpu.SemaphoreType.DMA,
    ],
)
def _sc_scatter(newm_hbm, ts_hbm, srcr_hbm, winr_hbm, mem_ref, lut_ref,
                sidx, widx, rows, tsv, sem):
  wid = _wid()
  pltpu.sync_copy(srcr_hbm.at[wid], sidx)
  pltpu.sync_copy(winr_hbm.at[wid], widx)
  for c in range(NCH):
    pltpu.async_copy(newm_hbm.at[widx.at[c]], rows, sem).wait()
    pltpu.async_copy(rows, mem_ref.at[sidx.at[c]], sem).wait()
    pltpu.async_copy(ts_hbm.at[widx.at[c]], tsv, sem).wait()
    pltpu.async_copy(tsv, lut_ref.at[sidx.at[c]], sem).wait()


TB = 2048


def _gru_body(dt_ref, s_ref, d_ref, ef_ref, tw_ref, tb_ref,
              wih_ref, whh_ref, bih_ref, bhh_ref, out_ref):
  s = s_ref[...]
  te = jnp.cos(dt_ref[...] * tw_ref[...] + tb_ref[...])
  wih = wih_ref[...]
  f32 = jnp.float32
  gi = (jnp.dot(s, wih[0:128], preferred_element_type=f32)
        + jnp.dot(d_ref[...], wih[128:256], preferred_element_type=f32)
        + jnp.dot(ef_ref[...], wih[256:272], preferred_element_type=f32)
        + jnp.dot(te, wih[272:288], preferred_element_type=f32)
        + bih_ref[...])
  gh = jnp.dot(s, whh_ref[...], preferred_element_type=f32) + bhh_ref[...]
  r = jax.nn.sigmoid(gi[:, 0:128] + gh[:, 0:128])
  z = jax.nn.sigmoid(gi[:, 128:256] + gh[:, 128:256])
  n = jnp.tanh(gi[:, 256:384] + r * gh[:, 256:384])
  out_ref[...] = (1.0 - z) * n + z * s


_gru_call = pl.pallas_call(
    _gru_body,
    grid=(B_EV // TB,),
    in_specs=[
        pl.BlockSpec((TB, 1), lambda i: (i, 0)),
        pl.BlockSpec((TB, D_MEM), lambda i: (i, 0)),
        pl.BlockSpec((TB, D_MEM), lambda i: (i, 0)),
        pl.BlockSpec((TB, D_EDGE), lambda i: (i, 0)),
        pl.BlockSpec((1, D_TIME), lambda i: (0, 0)),
        pl.BlockSpec((1, D_TIME), lambda i: (0, 0)),
        pl.BlockSpec((2 * D_MEM + D_EDGE + D_TIME, 3 * D_MEM), lambda i: (0, 0)),
        pl.BlockSpec((D_MEM, 3 * D_MEM), lambda i: (0, 0)),
        pl.BlockSpec((1, 3 * D_MEM), lambda i: (0, 0)),
        pl.BlockSpec((1, 3 * D_MEM), lambda i: (0, 0)),
    ],
    out_specs=pl.BlockSpec((TB, D_MEM), lambda i: (i, 0)),
    out_shape=jax.ShapeDtypeStruct((B_EV, D_MEM), jnp.float32),
)


def kernel(memory, last_update_t, src, dst, edge_feat, timestamps,
           time_w, time_b, w_ih, w_hh, b_ih, b_hh):
  src_r = src.astype(jnp.int32).reshape(NW, NCH, CH)
  dst_r = dst.astype(jnp.int32).reshape(NW, NCH, CH)

  src_mem, dst_mem, lut_src = _sc_gather(memory, last_update_t, src_r, dst_r)

  dt = (timestamps - lut_src).reshape(B_EV, 1)
  new_mem = _gru_call(dt, src_mem, dst_mem, edge_feat,
                      time_w.reshape(1, D_TIME), time_b.reshape(1, D_TIME),
                      w_ih.T, w_hh.T,
                      b_ih.reshape(1, 3 * D_MEM), b_hh.reshape(1, 3 * D_MEM))

  iota = jnp.arange(B_EV, dtype=jnp.int32)
  win = jnp.zeros((N_NODES,), jnp.int32).at[src].max(iota)[src]
  win_r = win.reshape(NW, NCH, CH)

  mem_ref = jax.new_ref(memory)
  lut_ref = jax.new_ref(last_update_t)
  _sc_scatter(new_mem, timestamps, src_r, win_r, mem_ref, lut_ref)
  return mem_ref[...], lut_ref[...]
